# baseline (speedup 1.0000x reference)
import functools

import jax
import jax.numpy as jnp
import numpy as np
from jax.experimental import pallas as pl

_MASK_RATIO = 0.15
_N_R = [50000, 100000, 200000, 300000]
_D_R = [64, 64, 32, 32]
_GRID = 20
_CLN = [2560, 5120, 10112, 15104]
_TOPO_E = [800000] * 8 + [200000, 200000, 400000, 400000, 600000, 600000]
_TOPO_BLK = {800000: 40960, 200000: 10240, 400000: 20480, 600000: 30720}


def _tf2x32_raw(k1, k2, x0, x1):
    rot0 = (13, 15, 26, 6)
    rot1 = (17, 29, 16, 24)
    ks0 = np.uint32(k1)
    ks1 = np.uint32(k2)
    ks2 = ks0 ^ ks1 ^ np.uint32(0x1BD11BDA)
    x0 = x0.astype(np.uint32) + ks0
    x1 = x1.astype(np.uint32) + ks1

    def rounds(a, b, rots):
        for r in rots:
            a = a + b
            b = (b << np.uint32(r)) | (b >> np.uint32(32 - r))
            b = a ^ b
        return a, b

    for rots, ka, kb, c in ((rot0, ks1, ks2, 1), (rot1, ks2, ks0, 2),
                            (rot0, ks0, ks1, 3), (rot1, ks1, ks2, 4),
                            (rot0, ks2, ks0, 5)):
        x0, x1 = rounds(x0, x1, rots)
        x0 = x0 + ka
        x1 = x1 + kb + np.uint32(c)
    return x0, x1


def _fold_in(key, data):
    x0, x1 = _tf2x32_raw(key[0], key[1],
                         np.zeros(1, np.uint32), np.full(1, data, np.uint32))
    return np.array([x0[0], x1[0]], np.uint32)


def _split2(key):
    b1, b2 = _tf2x32_raw(key[0], key[1],
                         np.zeros(2, np.uint32), np.arange(2, dtype=np.uint32))
    return (np.array([b1[0], b2[0]], np.uint32),
            np.array([b1[1], b2[1]], np.uint32))


def _np_permutation(key, n):
    num_rounds = int(np.ceil(3 * np.log(max(1, n)) / np.log(np.iinfo(np.uint32).max)))
    x = np.arange(n, dtype=np.int32)
    for _ in range(num_rounds):
        key, subkey = _split2(key)
        b1, b2 = _tf2x32_raw(subkey[0], subkey[1],
                             np.zeros(n, np.uint32), np.arange(n, dtype=np.uint32))
        x = x[np.argsort(b1 ^ b2, kind="stable")]
    return x


@functools.cache
def _masks():
    key = np.array([0, 0], np.uint32)
    out = []
    for r, n in enumerate(_N_R):
        n_mask = max(1, int(n * _MASK_RATIO))
        perm = _np_permutation(_fold_in(key, r), n)[:n_mask]
        m = np.zeros((n,), dtype=np.bool_)
        m[perm] = True
        out.append(m)
    return out


def _fused_kernel(*refs):
    m = refs[0:4]
    t = refs[4:8]
    f = refs[8:12]
    tin = refs[12:26]
    o = refs[26:30]
    tout = refs[30:44]
    for r in range(4):
        o[r][...] = jnp.where(m[r][...] > 0, t[r][...], f[r][...])
    for k in range(14):
        tout[k][...] = tin[k][...]


@functools.cache
def _fused_call():
    topo_specs = [pl.BlockSpec((_TOPO_BLK[e],), lambda i: (i,))
                  for e in _TOPO_E]
    in_specs = (
        [pl.BlockSpec((1, _CLN[r]), lambda i, r=r: (0, i)) for r in range(4)]
        + [pl.BlockSpec((_D_R[r], 1), lambda i: (0, 0)) for r in range(4)]
        + [pl.BlockSpec((_D_R[r], _CLN[r]), lambda i, r=r: (0, i))
           for r in range(4)]
        + topo_specs
    )
    out_specs = ([pl.BlockSpec((_D_R[r], _CLN[r]), lambda i, r=r: (0, i))
                  for r in range(4)]
                 + topo_specs)
    out_shape = ([jax.ShapeDtypeStruct((d, n), jnp.float32)
                  for n, d in zip(_N_R, _D_R)]
                 + [jax.ShapeDtypeStruct((e,), jnp.int32) for e in _TOPO_E])
    return pl.pallas_call(
        _fused_kernel,
        grid=(_GRID,),
        in_specs=in_specs,
        out_specs=out_specs,
        out_shape=out_shape,
    )


def kernel(feat0, feat1, feat2, feat3, mask_token0, mask_token1, mask_token2, mask_token3, nbr0_src, nbr0_dst, nbr1_src, nbr1_dst, nbr2_src, nbr2_dst, nbr3_src, nbr3_dst, inc_01_edge, inc_01_node, inc_12_bend, inc_12_edge, inc_23_torsion, inc_23_bend):
    feats = [feat0, feat1, feat2, feat3]
    tokens = [mask_token0, mask_token1, mask_token2, mask_token3]
    topo = [nbr0_src, nbr0_dst, nbr1_src, nbr1_dst, nbr2_src, nbr2_dst, nbr3_src, nbr3_dst, inc_01_edge, inc_01_node, inc_12_bend, inc_12_edge, inc_23_torsion, inc_23_bend]
    masks_np = _masks()

    mask_f32 = [jnp.asarray(m.astype(np.float32).reshape(1, -1))
                for m in masks_np]
    tok2d = [t.astype(jnp.float32).reshape(-1, 1) for t in tokens]
    feats_t = [f.T for f in feats]

    outs = _fused_call()(*mask_f32, *tok2d, *feats_t, *topo)
    masked_feats = [o.T for o in outs[0:4]]
    masked_topo = outs[4:18]
    masks = [jnp.asarray(m) for m in masks_np]
    return (*masked_feats, *masks, *masked_topo)


# speedup vs baseline: 1.0041x; 1.0041x over previous
import functools

import jax
import jax.numpy as jnp
import numpy as np
from jax.experimental import pallas as pl

_MASK_RATIO = 0.15
_N_R = [50000, 100000, 200000, 300000]
_D_R = [64, 64, 32, 32]
_GRID = 13
_CLN = [3968, 7808, 15488, 23168]
_TOPO_E = [800000] * 8 + [200000, 200000, 400000, 400000, 600000, 600000]
_TOPO_BLK = {800000: 62464, 200000: 16384, 400000: 31744, 600000: 47104}


def _tf2x32_raw(k1, k2, x0, x1):
    rot0 = (13, 15, 26, 6)
    rot1 = (17, 29, 16, 24)
    ks0 = np.uint32(k1)
    ks1 = np.uint32(k2)
    ks2 = ks0 ^ ks1 ^ np.uint32(0x1BD11BDA)
    x0 = x0.astype(np.uint32) + ks0
    x1 = x1.astype(np.uint32) + ks1

    def rounds(a, b, rots):
        for r in rots:
            a = a + b
            b = (b << np.uint32(r)) | (b >> np.uint32(32 - r))
            b = a ^ b
        return a, b

    for rots, ka, kb, c in ((rot0, ks1, ks2, 1), (rot1, ks2, ks0, 2),
                            (rot0, ks0, ks1, 3), (rot1, ks1, ks2, 4),
                            (rot0, ks2, ks0, 5)):
        x0, x1 = rounds(x0, x1, rots)
        x0 = x0 + ka
        x1 = x1 + kb + np.uint32(c)
    return x0, x1


def _fold_in(key, data):
    x0, x1 = _tf2x32_raw(key[0], key[1],
                         np.zeros(1, np.uint32), np.full(1, data, np.uint32))
    return np.array([x0[0], x1[0]], np.uint32)


def _split2(key):
    b1, b2 = _tf2x32_raw(key[0], key[1],
                         np.zeros(2, np.uint32), np.arange(2, dtype=np.uint32))
    return (np.array([b1[0], b2[0]], np.uint32),
            np.array([b1[1], b2[1]], np.uint32))


def _np_permutation(key, n):
    num_rounds = int(np.ceil(3 * np.log(max(1, n)) / np.log(np.iinfo(np.uint32).max)))
    x = np.arange(n, dtype=np.int32)
    for _ in range(num_rounds):
        key, subkey = _split2(key)
        b1, b2 = _tf2x32_raw(subkey[0], subkey[1],
                             np.zeros(n, np.uint32), np.arange(n, dtype=np.uint32))
        x = x[np.argsort(b1 ^ b2, kind="stable")]
    return x


@functools.cache
def _masks():
    key = np.array([0, 0], np.uint32)
    out = []
    for r, n in enumerate(_N_R):
        n_mask = max(1, int(n * _MASK_RATIO))
        perm = _np_permutation(_fold_in(key, r), n)[:n_mask]
        m = np.zeros((n,), dtype=np.bool_)
        m[perm] = True
        out.append(m)
    return out


def _fused_kernel(*refs):
    m = refs[0:4]
    t = refs[4:8]
    f = refs[8:12]
    tin = refs[12:26]
    o = refs[26:30]
    tout = refs[30:44]
    for r in range(4):
        o[r][...] = jnp.where(m[r][...] > 0, t[r][...], f[r][...])
    for k in range(14):
        tout[k][...] = tin[k][...]


@functools.cache
def _fused_call():
    topo_specs = [pl.BlockSpec((_TOPO_BLK[e],), lambda i: (i,))
                  for e in _TOPO_E]
    in_specs = (
        [pl.BlockSpec((1, _CLN[r]), lambda i, r=r: (0, i)) for r in range(4)]
        + [pl.BlockSpec((_D_R[r], 1), lambda i: (0, 0)) for r in range(4)]
        + [pl.BlockSpec((_D_R[r], _CLN[r]), lambda i, r=r: (0, i))
           for r in range(4)]
        + topo_specs
    )
    out_specs = ([pl.BlockSpec((_D_R[r], _CLN[r]), lambda i, r=r: (0, i))
                  for r in range(4)]
                 + topo_specs)
    out_shape = ([jax.ShapeDtypeStruct((d, n), jnp.float32)
                  for n, d in zip(_N_R, _D_R)]
                 + [jax.ShapeDtypeStruct((e,), jnp.int32) for e in _TOPO_E])
    return pl.pallas_call(
        _fused_kernel,
        grid=(_GRID,),
        in_specs=in_specs,
        out_specs=out_specs,
        out_shape=out_shape,
    )


def kernel(feat0, feat1, feat2, feat3, mask_token0, mask_token1, mask_token2, mask_token3, nbr0_src, nbr0_dst, nbr1_src, nbr1_dst, nbr2_src, nbr2_dst, nbr3_src, nbr3_dst, inc_01_edge, inc_01_node, inc_12_bend, inc_12_edge, inc_23_torsion, inc_23_bend):
    feats = [feat0, feat1, feat2, feat3]
    tokens = [mask_token0, mask_token1, mask_token2, mask_token3]
    topo = [nbr0_src, nbr0_dst, nbr1_src, nbr1_dst, nbr2_src, nbr2_dst, nbr3_src, nbr3_dst, inc_01_edge, inc_01_node, inc_12_bend, inc_12_edge, inc_23_torsion, inc_23_bend]
    masks_np = _masks()

    mask_f32 = [jnp.asarray(m.astype(np.float32).reshape(1, -1))
                for m in masks_np]
    tok2d = [t.astype(jnp.float32).reshape(-1, 1) for t in tokens]
    feats_t = [f.T for f in feats]

    outs = _fused_call()(*mask_f32, *tok2d, *feats_t, *topo)
    masked_feats = [o.T for o in outs[0:4]]
    masked_topo = outs[4:18]
    masks = [jnp.asarray(m) for m in masks_np]
    return (*masked_feats, *masks, *masked_topo)
